# baseline (speedup 1.0000x reference)
import functools

import jax
import jax.numpy as jnp
from jax import lax
from jax.experimental import pallas as pl
from jax.experimental.pallas import tpu as pltpu
from jax.experimental.pallas import tpu_sc as plsc

N = 8192
C = 32
NUM_RBF = 32
M0 = 512
M1 = 128
K = 16
R0 = 0.2
R1 = 0.4

_BIGF = 1e10
_REMOVED = 3e10


def _iota2(shape, dim):
    return lax.broadcasted_iota(jnp.int32, shape, dim)


def _silu(x):
    return x * (1.0 / (1.0 + jnp.exp(-x)))


def _prep_body(pos_ref, aug_ref):
    p = pos_ref[...]
    x = p[:, 0:1]
    y = p[:, 1:2]
    z = p[:, 2:3]
    s0 = jnp.sqrt((x * x + y * y) + z * z)
    aug_ref[...] = jnp.concatenate(
        [s0, x, y, z, jnp.zeros((N, 124), jnp.float32)], axis=1)


def _prep(pos):
    return pl.pallas_call(
        _prep_body,
        out_shape=jax.ShapeDtypeStruct((N, 128), jnp.float32),
    )(pos)


def _fps_body(npts, nsel, px_ref, py_ref, pz_ref, ox_ref, oy_ref, oz_ref):
    px = px_ref[...]
    py = py_ref[...]
    pz = pz_ref[...]
    shape = px.shape
    cols = shape[1]
    lin = _iota2(shape, 0) * cols + _iota2(shape, 1)
    oshape = (8, nsel // 8)
    ocols = oshape[1]
    olin = _iota2(oshape, 0) * ocols + _iota2(oshape, 1)

    def body(i, carry):
        dist, cx, cy, cz, ax, ay, az = carry
        slot = olin == i
        ax = jnp.where(slot, cx, ax)
        ay = jnp.where(slot, cy, ay)
        az = jnp.where(slot, cz, az)
        dx = px - cx
        dy = py - cy
        dz = pz - cz
        d2 = (dx * dx + dy * dy) + dz * dz
        dist = jnp.minimum(dist, d2)
        mx = jnp.max(dist, axis=1, keepdims=True)
        mx = jnp.max(mx, axis=0, keepdims=True)
        t = jnp.where(dist == mx, lin, npts)
        t = jnp.min(t, axis=1, keepdims=True)
        cur = jnp.min(t, axis=0, keepdims=True)
        oh = lin == cur
        zf = jnp.zeros(shape, jnp.float32)
        cx = jnp.sum(jnp.where(oh, px, zf), axis=1, keepdims=True)
        cx = jnp.sum(cx, axis=0, keepdims=True)
        cy = jnp.sum(jnp.where(oh, py, zf), axis=1, keepdims=True)
        cy = jnp.sum(cy, axis=0, keepdims=True)
        cz = jnp.sum(jnp.where(oh, pz, zf), axis=1, keepdims=True)
        cz = jnp.sum(cz, axis=0, keepdims=True)
        return dist, cx, cy, cz, ax, ay, az

    dist0 = jnp.full(shape, 1e30, jnp.float32)
    c0x = px[0:1, 0:1]
    c0y = py[0:1, 0:1]
    c0z = pz[0:1, 0:1]
    zo = jnp.zeros(oshape, jnp.float32)
    _, _, _, _, ax, ay, az = lax.fori_loop(
        0, nsel, body, (dist0, c0x, c0y, c0z, zo, zo, zo))
    ox_ref[...] = ax
    oy_ref[...] = ay
    oz_ref[...] = az


def _fps(px, py, pz, nsel):
    npts = px.shape[0] * px.shape[1]
    f = jax.ShapeDtypeStruct((8, nsel // 8), jnp.float32)
    return pl.pallas_call(
        functools.partial(_fps_body, npts, nsel),
        out_shape=(f, f, f),
    )(px, py, pz)


def _bq_body(npts, r2, cx_ref, cy_ref, cz_ref, px_ref, py_ref, pz_ref,
             idx_ref, d2_ref):
    cx = cx_ref[...]
    cy = cy_ref[...]
    cz = cz_ref[...]
    px = px_ref[...]
    py = py_ref[...]
    pz = pz_ref[...]
    dx = cx - px
    dy = cy - py
    dz = cz - pz
    d2 = (dx * dx + dy * dy) + dz * dz
    d = jnp.where(d2 <= r2, d2, _BIGF)
    nb = cx.shape[0]
    lin = jnp.broadcast_to(_iota2((1, npts), 1), d.shape)
    lane = _iota2((nb, K), 1)
    vacc = jnp.zeros((nb, K), jnp.float32)
    iacc = jnp.zeros((nb, K), jnp.int32)
    for e in range(K):
        m = jnp.min(d, axis=1, keepdims=True)
        t = jnp.where(d == m, lin, npts)
        ix = jnp.min(t, axis=1, keepdims=True)
        vacc = jnp.where(lane == e, m, vacc)
        iacc = jnp.where(lane == e, ix, iacc)
        d = jnp.where(lin == ix, _REMOVED, d)
    valid = vacc < 1e9
    i0 = iacc[:, 0:1]
    v0 = vacc[:, 0:1]
    idx_ref[...] = jnp.where(valid, iacc, i0)
    d2_ref[...] = jnp.where(valid, vacc, v0)


def _ball_query(cx, cy, cz, px, py, pz, radius, block):
    nc = cx.shape[0]
    npts = px.shape[1]
    grid = nc // block
    cspec = pl.BlockSpec((block, 1), lambda b: (b, 0))
    pspec = pl.BlockSpec((1, npts), lambda b: (0, 0))
    ospec = pl.BlockSpec((block, K), lambda b: (b, 0))
    return pl.pallas_call(
        functools.partial(_bq_body, npts, radius * radius),
        grid=(grid,),
        in_specs=[cspec, cspec, cspec, pspec, pspec, pspec],
        out_specs=(ospec, ospec),
        out_shape=(jax.ShapeDtypeStruct((nc, K), jnp.int32),
                   jax.ShapeDtypeStruct((nc, K), jnp.float32)),
    )(cx, cy, cz, px, py, pz)


_SC_NC = 2
_SC_NS = 16
_SC_NW = _SC_NC * _SC_NS
_SC_BPW = N // _SC_NW


def _sc_gather_body(idx_hbm, tab_hbm, out_hbm, idx_v, rows_v, sem):
    wid = lax.axis_index("s") * _SC_NC + lax.axis_index("c")
    base = wid * _SC_BPW
    pltpu.sync_copy(idx_hbm.at[pl.ds(base, _SC_BPW)], idx_v)
    pltpu.async_copy(tab_hbm.at[idx_v], rows_v, sem).wait()
    pltpu.sync_copy(rows_v, out_hbm.at[pl.ds(base, _SC_BPW)])


def _sc_gather(idx, tab):
    mesh = plsc.VectorSubcoreMesh(core_axis_name="c", subcore_axis_name="s")
    k = pl.kernel(
        _sc_gather_body,
        mesh=mesh,
        out_type=jax.ShapeDtypeStruct((N, 128), jnp.float32),
        scratch_types=[
            pltpu.VMEM((_SC_BPW,), jnp.int32),
            pltpu.VMEM((_SC_BPW, 128), jnp.float32),
            pltpu.SemaphoreType.DMA,
        ],
    )
    return k(idx, tab)


_ROWS_PER_TILE = M0 // _SC_NW
_NCHUNK = N // 16
_BIG2 = 4e10


def _sc_bq_body(cx_hbm, cy_hbm, cz_hbm, px_hbm, py_hbm, pz_hbm,
                outi_hbm, outd_hbm,
                pxv, pyv, pzv, cxv, cyv, czv, d2b, ib, oiv, odv, sem):
    del sem
    wid = lax.axis_index("s") * _SC_NC + lax.axis_index("c")
    base = wid * _ROWS_PER_TILE
    pltpu.sync_copy(px_hbm, pxv)
    pltpu.sync_copy(py_hbm, pyv)
    pltpu.sync_copy(pz_hbm, pzv)
    pltpu.sync_copy(cx_hbm.at[pl.ds(base, _ROWS_PER_TILE)], cxv)
    pltpu.sync_copy(cy_hbm.at[pl.ds(base, _ROWS_PER_TILE)], cyv)
    pltpu.sync_copy(cz_hbm.at[pl.ds(base, _ROWS_PER_TILE)], czv)

    lane = lax.iota(jnp.int32, 16)
    r2 = jnp.float32(R0 * R0)

    cxall = cxv[...]
    cyall = cyv[...]
    czall = czv[...]

    def _splat(vec, s):
        return vec.at[jnp.full((16,), s, jnp.int32)].get(
            mode="promise_in_bounds")

    def row_body(r, acc):
        cxs = _splat(cxall, r)
        cys = _splat(cyall, r)
        czs = _splat(czall, r)

        def compact(j, cnt):
            o = j * 16
            x = pxv[pl.ds(o, 16)] - cxs
            y = pyv[pl.ds(o, 16)] - cys
            z = pzv[pl.ds(o, 16)] - czs
            d2 = (x * x + y * y) + z * z
            msk = d2 <= r2
            pc = plsc.cumsum(msk.astype(jnp.int32))
            tgt = cnt + pc - 1
            plsc.store_scatter(d2b, [tgt], d2, mask=msk)
            plsc.store_scatter(ib, [tgt], o + lane, mask=msk)
            return cnt + plsc.all_reduce_population_count(msk)

        cnt = lax.fori_loop(0, _NCHUNK, compact,
                            jnp.zeros((16,), jnp.int32))
        nl = jnp.max(cnt)
        nch = (nl + 15) // 16

        def extract(e, carry):
            v16, i16 = carry

            def scan(c, bc):
                bv, bp, bi = bc
                o = c * 16
                g = o + lane
                v = d2b[pl.ds(o, 16)]
                i = ib[pl.ds(o, 16)]
                v = jnp.where(g < nl, v, _BIG2)
                lt = v < bv
                return (jnp.where(lt, v, bv), jnp.where(lt, g, bp),
                        jnp.where(lt, i, bi))

            bv, bp, bi = lax.fori_loop(
                0, nch, scan,
                (jnp.full((16,), _BIG2), jnp.zeros((16,), jnp.int32),
                 jnp.zeros((16,), jnp.int32)))
            m = jnp.min(bv)
            p = jnp.min(jnp.where(bv == m, bp, jnp.int32(N)))
            found = m < 1e9
            p = jnp.where(found, p, 0)
            oi = jnp.min(jnp.where((bv == m) & (bp == p), bi, jnp.int32(N)))
            sel = lane == e
            v16 = jnp.where(sel & found, jnp.full((16,), m), v16)
            i16 = jnp.where(sel & found, jnp.full((16,), oi, jnp.int32), i16)
            pch = (p // 16) * 16
            pin = p - pch
            ch = d2b[pl.ds(pch, 16)]
            ch = jnp.where(found & (lane == pin), _BIG2, ch)
            d2b[pl.ds(pch, 16)] = ch
            return v16, i16

        v16, i16 = lax.fori_loop(
            0, K, extract,
            (jnp.full((16,), _BIGF), jnp.zeros((16,), jnp.int32)))
        valid = v16 < 1e9
        v0 = _splat(v16, 0)
        i0 = _splat(i16, 0)
        odv[pl.ds(r * K, K)] = jnp.where(valid, v16, v0)
        oiv[pl.ds(r * K, K)] = jnp.where(valid, i16, i0)
        return acc

    lax.fori_loop(0, _ROWS_PER_TILE, row_body, jnp.int32(0))
    nv = _ROWS_PER_TILE * K
    pltpu.sync_copy(oiv, outi_hbm.at[pl.ds(base * K, nv)])
    pltpu.sync_copy(odv, outd_hbm.at[pl.ds(base * K, nv)])


def _sc_ball_query0(cx, cy, cz, px, py, pz):
    mesh = plsc.VectorSubcoreMesh(core_axis_name="c", subcore_axis_name="s")
    k = pl.kernel(
        _sc_bq_body,
        mesh=mesh,
        out_type=(jax.ShapeDtypeStruct((M0 * K,), jnp.int32),
                  jax.ShapeDtypeStruct((M0 * K,), jnp.float32)),
        scratch_types=[
            pltpu.VMEM((N,), jnp.float32),
            pltpu.VMEM((N,), jnp.float32),
            pltpu.VMEM((N,), jnp.float32),
            pltpu.VMEM((_ROWS_PER_TILE,), jnp.float32),
            pltpu.VMEM((_ROWS_PER_TILE,), jnp.float32),
            pltpu.VMEM((_ROWS_PER_TILE,), jnp.float32),
            pltpu.VMEM((N,), jnp.float32),
            pltpu.VMEM((N,), jnp.int32),
            pltpu.VMEM((_ROWS_PER_TILE * K,), jnp.int32),
            pltpu.VMEM((_ROWS_PER_TILE * K,), jnp.float32),
            pltpu.SemaphoreType.DMA,
        ],
    )
    oi, od = k(cx, cy, cz, px, py, pz)
    return oi.reshape(M0, K), od.reshape(M0, K)


def _rbf_h(d2, cutoff, wr1, wr2):
    dist = jnp.maximum(jnp.sqrt(d2), 1e-8)
    mu = _iota2((1, NUM_RBF), 1).astype(jnp.float32) * (
        cutoff / (NUM_RBF - 1))
    beta = (NUM_RBF / cutoff) ** 2
    dd = dist - mu
    rbf = jnp.exp(-beta * (dd * dd))
    a = _silu(jnp.dot(rbf, wr1, preferred_element_type=jnp.float32, precision=jax.lax.Precision.HIGHEST))
    return jnp.dot(a, wr2, preferred_element_type=jnp.float32, precision=jax.lax.Precision.HIGHEST)


def _kmax(t, m):
    acc = t[0:m, :]
    for k in range(1, K):
        acc = jnp.maximum(acc, t[k * m:(k + 1) * m, :])
    return acc


def _stage0_body(g_ref, d2_ref, wr1_ref, wr2_ref, w0s_ref, w0v_ref,
                 wms_ref, wmv_ref, out_ref):
    g = g_ref[...]
    h = _rbf_h(d2_ref[...], R0, wr1_ref[...], wr2_ref[...])
    w0s = w0s_ref[...]
    w0v = w0v_ref[...]
    fs = _kmax(h * g[:, 0:1] * w0s, M0)
    fv0 = _kmax(h * g[:, 1:2] * w0v, M0)
    fv1 = _kmax(h * g[:, 2:3] * w0v, M0)
    fv2 = _kmax(h * g[:, 3:4] * w0v, M0)
    wms = wms_ref[...]
    wmv = wmv_ref[...]
    out_ref[...] = jnp.concatenate([
        jnp.dot(fs, wms, preferred_element_type=jnp.float32, precision=jax.lax.Precision.HIGHEST),
        jnp.dot(fv0, wmv, preferred_element_type=jnp.float32, precision=jax.lax.Precision.HIGHEST),
        jnp.dot(fv1, wmv, preferred_element_type=jnp.float32, precision=jax.lax.Precision.HIGHEST),
        jnp.dot(fv2, wmv, preferred_element_type=jnp.float32, precision=jax.lax.Precision.HIGHEST),
    ], axis=1)


def _stage0(gn, d2, p):
    return pl.pallas_call(
        _stage0_body,
        out_shape=jax.ShapeDtypeStruct((M0, 4 * C), jnp.float32),
    )(gn, d2, p['Wr1_0'], p['Wr2_0'], p['W0_s'], p['W0_v'],
      p['Wms_0'], p['Wmv_0'])


def _stage1_body(fb_ref, idx_ref, d2_ref, wr1_ref, wr2_ref, wms_ref,
                 wmv_ref, wc1_ref, bc1_ref, wc2_ref, bc2_ref, wc3_ref,
                 bc3_ref, out_ref):
    fb = fb_ref[...]
    idx = idx_ref[...]
    oh = (idx == _iota2((1, M0), 1)).astype(jnp.float32)
    g = jnp.dot(oh, fb, preferred_element_type=jnp.float32, precision=jax.lax.Precision.HIGHEST)
    h = _rbf_h(d2_ref[...], R1, wr1_ref[...], wr2_ref[...])
    fs = _kmax(h * g[:, 0:C], M1)
    fv0 = _kmax(h * g[:, C:2 * C], M1)
    fv1 = _kmax(h * g[:, 2 * C:3 * C], M1)
    fv2 = _kmax(h * g[:, 3 * C:4 * C], M1)
    wms = wms_ref[...]
    wmv = wmv_ref[...]
    fs = jnp.dot(fs, wms, preferred_element_type=jnp.float32, precision=jax.lax.Precision.HIGHEST)
    fv0 = jnp.dot(fv0, wmv, preferred_element_type=jnp.float32, precision=jax.lax.Precision.HIGHEST)
    fv1 = jnp.dot(fv1, wmv, preferred_element_type=jnp.float32, precision=jax.lax.Precision.HIGHEST)
    fv2 = jnp.dot(fv2, wmv, preferred_element_type=jnp.float32, precision=jax.lax.Precision.HIGHEST)
    vn = jnp.sqrt((fv0 * fv0 + fv1 * fv1) + fv2 * fv2)
    inv = jnp.concatenate([fs, vn], axis=1)
    gmax = jnp.max(inv, axis=0, keepdims=True)
    h1 = _silu(jnp.dot(gmax, wc1_ref[...],
                       preferred_element_type=jnp.float32, precision=jax.lax.Precision.HIGHEST) + bc1_ref[...])
    m = jnp.mean(h1, axis=1, keepdims=True)
    v = jnp.mean((h1 - m) * (h1 - m), axis=1, keepdims=True)
    h1 = (h1 - m) / jnp.sqrt(v + 1e-5)
    h2 = _silu(jnp.dot(h1, wc2_ref[...],
                       preferred_element_type=jnp.float32, precision=jax.lax.Precision.HIGHEST) + bc2_ref[...])
    m = jnp.mean(h2, axis=1, keepdims=True)
    v = jnp.mean((h2 - m) * (h2 - m), axis=1, keepdims=True)
    h2 = (h2 - m) / jnp.sqrt(v + 1e-5)
    out_ref[...] = jnp.dot(h2, wc3_ref[...],
                           preferred_element_type=jnp.float32, precision=jax.lax.Precision.HIGHEST) + bc3_ref[...]


def _stage1(fb, idx, d2, p):
    return pl.pallas_call(
        _stage1_body,
        out_shape=jax.ShapeDtypeStruct((1, 40), jnp.float32),
    )(fb, idx, d2, p['Wr1_1'], p['Wr2_1'], p['Wms_1'], p['Wmv_1'],
      p['Wc1'], p['bc1'].reshape(1, -1), p['Wc2'], p['bc2'].reshape(1, -1),
      p['Wc3'], p['bc3'].reshape(1, -1))


def kernel(pos, params):
    posT = pos.T
    px8 = posT[0].reshape(8, N // 8)
    py8 = posT[1].reshape(8, N // 8)
    pz8 = posT[2].reshape(8, N // 8)
    px1 = posT[0].reshape(1, N)
    py1 = posT[1].reshape(1, N)
    pz1 = posT[2].reshape(1, N)

    aug = _prep(pos)

    c0x, c0y, c0z = _fps(px8, py8, pz8, M0)
    idx0, d20 = _ball_query(
        c0x.reshape(M0, 1), c0y.reshape(M0, 1), c0z.reshape(M0, 1),
        px1, py1, pz1, R0, 64)

    idx0f = idx0.T.reshape(N)
    gn = _sc_gather(idx0f, aug)
    fb = _stage0(gn, d20.T.reshape(N, 1), params)

    c1x, c1y, c1z = _fps(c0x, c0y, c0z, M1)
    idx1, d21 = _ball_query(
        c1x.reshape(M1, 1), c1y.reshape(M1, 1), c1z.reshape(M1, 1),
        c0x.reshape(1, M0), c0y.reshape(1, M0), c0z.reshape(1, M0),
        R1, 64)

    out = _stage1(fb, idx1.T.reshape(K * M1, 1), d21.T.reshape(K * M1, 1),
                  params)
    return out.reshape(40)


# speedup vs baseline: 1.0852x; 1.0852x over previous
import functools

import jax
import jax.numpy as jnp
from jax import lax
from jax.experimental import pallas as pl
from jax.experimental.pallas import tpu as pltpu
from jax.experimental.pallas import tpu_sc as plsc

N = 8192
C = 32
NUM_RBF = 32
M0 = 512
M1 = 128
K = 16
R0 = 0.2
R1 = 0.4

_BIGF = 1e10
_REMOVED = 3e10


def _iota2(shape, dim):
    return lax.broadcasted_iota(jnp.int32, shape, dim)


def _silu(x):
    return x * (1.0 / (1.0 + jnp.exp(-x)))


def _prep_body(pos_ref, aug_ref):
    p = pos_ref[...]
    x = p[:, 0:1]
    y = p[:, 1:2]
    z = p[:, 2:3]
    s0 = jnp.sqrt((x * x + y * y) + z * z)
    aug_ref[...] = jnp.concatenate(
        [s0, x, y, z, jnp.zeros((N, 124), jnp.float32)], axis=1)


def _prep(pos):
    return pl.pallas_call(
        _prep_body,
        out_shape=jax.ShapeDtypeStruct((N, 128), jnp.float32),
    )(pos)


def _fps_body(npts, nsel, px_ref, py_ref, pz_ref, ox_ref, oy_ref, oz_ref):
    px = px_ref[...]
    py = py_ref[...]
    pz = pz_ref[...]
    shape = px.shape
    cols = shape[1]
    lin = _iota2(shape, 0) * cols + _iota2(shape, 1)
    oshape = (8, nsel // 8)
    ocols = oshape[1]
    olin = _iota2(oshape, 0) * ocols + _iota2(oshape, 1)

    cat = jnp.concatenate([px, py, pz], axis=0)
    zc = jnp.zeros(cat.shape, jnp.float32)

    def step(i, carry):
        dist, cx, cy, cz, ax, ay, az = carry
        slot = olin == i
        ax = jnp.where(slot, cx, ax)
        ay = jnp.where(slot, cy, ay)
        az = jnp.where(slot, cz, az)
        dx = px - cx
        dy = py - cy
        dz = pz - cz
        d2 = (dx * dx + dy * dy) + dz * dz
        dist = jnp.minimum(dist, d2)
        mx = jnp.max(dist, axis=1, keepdims=True)
        mx = jnp.max(mx, axis=0, keepdims=True)
        t = jnp.where(dist == mx, lin, npts)
        t = jnp.min(t, axis=1, keepdims=True)
        cur = jnp.min(t, axis=0, keepdims=True)
        oh = lin == cur
        oh3 = jnp.concatenate([oh, oh, oh], axis=0)
        s1 = jnp.sum(jnp.where(oh3, cat, zc), axis=1, keepdims=True)
        cx = jnp.sum(s1[0:8], axis=0, keepdims=True)
        cy = jnp.sum(s1[8:16], axis=0, keepdims=True)
        cz = jnp.sum(s1[16:24], axis=0, keepdims=True)
        return dist, cx, cy, cz, ax, ay, az

    def body(i, carry):
        carry = step(2 * i, carry)
        return step(2 * i + 1, carry)

    dist0 = jnp.full(shape, 1e30, jnp.float32)
    c0x = px[0:1, 0:1]
    c0y = py[0:1, 0:1]
    c0z = pz[0:1, 0:1]
    zo = jnp.zeros(oshape, jnp.float32)
    _, _, _, _, ax, ay, az = lax.fori_loop(
        0, nsel // 2, body, (dist0, c0x, c0y, c0z, zo, zo, zo))
    ox_ref[...] = ax
    oy_ref[...] = ay
    oz_ref[...] = az


def _fps(px, py, pz, nsel):
    npts = px.shape[0] * px.shape[1]
    f = jax.ShapeDtypeStruct((8, nsel // 8), jnp.float32)
    return pl.pallas_call(
        functools.partial(_fps_body, npts, nsel),
        out_shape=(f, f, f),
    )(px, py, pz)


def _bq_body(npts, r2, cx_ref, cy_ref, cz_ref, px_ref, py_ref, pz_ref,
             idx_ref, d2_ref):
    cx = cx_ref[...]
    cy = cy_ref[...]
    cz = cz_ref[...]
    px = px_ref[...]
    py = py_ref[...]
    pz = pz_ref[...]
    dx = cx - px
    dy = cy - py
    dz = cz - pz
    d2 = (dx * dx + dy * dy) + dz * dz
    d = jnp.where(d2 <= r2, d2, _BIGF)
    nb = cx.shape[0]
    lin = jnp.broadcast_to(_iota2((1, npts), 1), d.shape)
    lane = _iota2((nb, K), 1)
    vacc = jnp.zeros((nb, K), jnp.float32)
    iacc = jnp.zeros((nb, K), jnp.int32)
    for e in range(K):
        m = jnp.min(d, axis=1, keepdims=True)
        t = jnp.where(d == m, lin, npts)
        ix = jnp.min(t, axis=1, keepdims=True)
        vacc = jnp.where(lane == e, m, vacc)
        iacc = jnp.where(lane == e, ix, iacc)
        d = jnp.where(lin == ix, _REMOVED, d)
    valid = vacc < 1e9
    i0 = iacc[:, 0:1]
    v0 = vacc[:, 0:1]
    idx_ref[...] = jnp.where(valid, iacc, i0)
    d2_ref[...] = jnp.where(valid, vacc, v0)


def _ball_query(cx, cy, cz, px, py, pz, radius, block):
    nc = cx.shape[0]
    npts = px.shape[1]
    grid = nc // block
    cspec = pl.BlockSpec((block, 1), lambda b: (b, 0))
    pspec = pl.BlockSpec((1, npts), lambda b: (0, 0))
    ospec = pl.BlockSpec((block, K), lambda b: (b, 0))
    return pl.pallas_call(
        functools.partial(_bq_body, npts, radius * radius),
        grid=(grid,),
        in_specs=[cspec, cspec, cspec, pspec, pspec, pspec],
        out_specs=(ospec, ospec),
        out_shape=(jax.ShapeDtypeStruct((nc, K), jnp.int32),
                   jax.ShapeDtypeStruct((nc, K), jnp.float32)),
    )(cx, cy, cz, px, py, pz)


_SC_NC = 2
_SC_NS = 16
_SC_NW = _SC_NC * _SC_NS
_SC_BPW = N // _SC_NW


def _sc_gather_body(idx_hbm, tab_hbm, out_hbm, idx_v, rows_v, sem):
    wid = lax.axis_index("s") * _SC_NC + lax.axis_index("c")
    base = wid * _SC_BPW
    pltpu.sync_copy(idx_hbm.at[pl.ds(base, _SC_BPW)], idx_v)
    pltpu.async_copy(tab_hbm.at[idx_v], rows_v, sem).wait()
    pltpu.sync_copy(rows_v, out_hbm.at[pl.ds(base, _SC_BPW)])


def _sc_gather(idx, tab):
    mesh = plsc.VectorSubcoreMesh(core_axis_name="c", subcore_axis_name="s")
    k = pl.kernel(
        _sc_gather_body,
        mesh=mesh,
        out_type=jax.ShapeDtypeStruct((N, 128), jnp.float32),
        scratch_types=[
            pltpu.VMEM((_SC_BPW,), jnp.int32),
            pltpu.VMEM((_SC_BPW, 128), jnp.float32),
            pltpu.SemaphoreType.DMA,
        ],
    )
    return k(idx, tab)


_ROWS_PER_TILE = M0 // _SC_NW
_NCHUNK = N // 16
_BIG2 = 4e10


def _sc_bq_body(cx_hbm, cy_hbm, cz_hbm, px_hbm, py_hbm, pz_hbm,
                outi_hbm, outd_hbm,
                pxv, pyv, pzv, cxv, cyv, czv, d2b, ib, oiv, odv, sem):
    del sem
    wid = lax.axis_index("s") * _SC_NC + lax.axis_index("c")
    base = wid * _ROWS_PER_TILE
    pltpu.sync_copy(px_hbm, pxv)
    pltpu.sync_copy(py_hbm, pyv)
    pltpu.sync_copy(pz_hbm, pzv)
    pltpu.sync_copy(cx_hbm.at[pl.ds(base, _ROWS_PER_TILE)], cxv)
    pltpu.sync_copy(cy_hbm.at[pl.ds(base, _ROWS_PER_TILE)], cyv)
    pltpu.sync_copy(cz_hbm.at[pl.ds(base, _ROWS_PER_TILE)], czv)

    lane = lax.iota(jnp.int32, 16)
    r2 = jnp.float32(R0 * R0)

    cxall = cxv[...]
    cyall = cyv[...]
    czall = czv[...]

    def _splat(vec, s):
        return vec.at[jnp.full((16,), s, jnp.int32)].get(
            mode="promise_in_bounds")

    def row_body(r, acc):
        cxs = _splat(cxall, r)
        cys = _splat(cyall, r)
        czs = _splat(czall, r)

        def compact(j, cnt):
            o = j * 16
            x = pxv[pl.ds(o, 16)] - cxs
            y = pyv[pl.ds(o, 16)] - cys
            z = pzv[pl.ds(o, 16)] - czs
            d2 = (x * x + y * y) + z * z
            msk = d2 <= r2
            pc = plsc.cumsum(msk.astype(jnp.int32))
            tgt = cnt + pc - 1
            plsc.store_scatter(d2b, [tgt], d2, mask=msk)
            plsc.store_scatter(ib, [tgt], o + lane, mask=msk)
            return cnt + plsc.all_reduce_population_count(msk)

        cnt = lax.fori_loop(0, _NCHUNK, compact,
                            jnp.zeros((16,), jnp.int32))
        nl = jnp.max(cnt)
        nch = (nl + 15) // 16

        def extract(e, carry):
            v16, i16 = carry

            def scan(c, bc):
                bv, bp, bi = bc
                o = c * 16
                g = o + lane
                v = d2b[pl.ds(o, 16)]
                i = ib[pl.ds(o, 16)]
                v = jnp.where(g < nl, v, _BIG2)
                lt = v < bv
                return (jnp.where(lt, v, bv), jnp.where(lt, g, bp),
                        jnp.where(lt, i, bi))

            bv, bp, bi = lax.fori_loop(
                0, nch, scan,
                (jnp.full((16,), _BIG2), jnp.zeros((16,), jnp.int32),
                 jnp.zeros((16,), jnp.int32)))
            m = jnp.min(bv)
            p = jnp.min(jnp.where(bv == m, bp, jnp.int32(N)))
            found = m < 1e9
            p = jnp.where(found, p, 0)
            oi = jnp.min(jnp.where((bv == m) & (bp == p), bi, jnp.int32(N)))
            sel = lane == e
            v16 = jnp.where(sel & found, jnp.full((16,), m), v16)
            i16 = jnp.where(sel & found, jnp.full((16,), oi, jnp.int32), i16)
            pch = (p // 16) * 16
            pin = p - pch
            ch = d2b[pl.ds(pch, 16)]
            ch = jnp.where(found & (lane == pin), _BIG2, ch)
            d2b[pl.ds(pch, 16)] = ch
            return v16, i16

        v16, i16 = lax.fori_loop(
            0, K, extract,
            (jnp.full((16,), _BIGF), jnp.zeros((16,), jnp.int32)))
        valid = v16 < 1e9
        v0 = _splat(v16, 0)
        i0 = _splat(i16, 0)
        odv[pl.ds(r * K, K)] = jnp.where(valid, v16, v0)
        oiv[pl.ds(r * K, K)] = jnp.where(valid, i16, i0)
        return acc

    lax.fori_loop(0, _ROWS_PER_TILE, row_body, jnp.int32(0))
    nv = _ROWS_PER_TILE * K
    pltpu.sync_copy(oiv, outi_hbm.at[pl.ds(base * K, nv)])
    pltpu.sync_copy(odv, outd_hbm.at[pl.ds(base * K, nv)])


def _sc_ball_query0(cx, cy, cz, px, py, pz):
    mesh = plsc.VectorSubcoreMesh(core_axis_name="c", subcore_axis_name="s")
    k = pl.kernel(
        _sc_bq_body,
        mesh=mesh,
        out_type=(jax.ShapeDtypeStruct((M0 * K,), jnp.int32),
                  jax.ShapeDtypeStruct((M0 * K,), jnp.float32)),
        scratch_types=[
            pltpu.VMEM((N,), jnp.float32),
            pltpu.VMEM((N,), jnp.float32),
            pltpu.VMEM((N,), jnp.float32),
            pltpu.VMEM((_ROWS_PER_TILE,), jnp.float32),
            pltpu.VMEM((_ROWS_PER_TILE,), jnp.float32),
            pltpu.VMEM((_ROWS_PER_TILE,), jnp.float32),
            pltpu.VMEM((N,), jnp.float32),
            pltpu.VMEM((N,), jnp.int32),
            pltpu.VMEM((_ROWS_PER_TILE * K,), jnp.int32),
            pltpu.VMEM((_ROWS_PER_TILE * K,), jnp.float32),
            pltpu.SemaphoreType.DMA,
        ],
    )
    oi, od = k(cx, cy, cz, px, py, pz)
    return oi.reshape(M0, K), od.reshape(M0, K)


def _rbf_h(d2, cutoff, wr1, wr2):
    dist = jnp.maximum(jnp.sqrt(d2), 1e-8)
    mu = _iota2((1, NUM_RBF), 1).astype(jnp.float32) * (
        cutoff / (NUM_RBF - 1))
    beta = (NUM_RBF / cutoff) ** 2
    dd = dist - mu
    rbf = jnp.exp(-beta * (dd * dd))
    a = _silu(jnp.dot(rbf, wr1, preferred_element_type=jnp.float32))
    return jnp.dot(a, wr2, preferred_element_type=jnp.float32)


def _kmax(t, m):
    acc = t[0:m, :]
    for k in range(1, K):
        acc = jnp.maximum(acc, t[k * m:(k + 1) * m, :])
    return acc


def _stage0_body(g_ref, d2_ref, wr1_ref, wr2_ref, w0s_ref, w0v_ref,
                 wms_ref, wmv_ref, out_ref):
    g = g_ref[...]
    h = _rbf_h(d2_ref[...], R0, wr1_ref[...], wr2_ref[...])
    w0s = w0s_ref[...]
    w0v = w0v_ref[...]
    fs = _kmax(h * g[:, 0:1] * w0s, M0)
    fv0 = _kmax(h * g[:, 1:2] * w0v, M0)
    fv1 = _kmax(h * g[:, 2:3] * w0v, M0)
    fv2 = _kmax(h * g[:, 3:4] * w0v, M0)
    wms = wms_ref[...]
    wmv = wmv_ref[...]
    out_ref[...] = jnp.concatenate([
        jnp.dot(fs, wms, preferred_element_type=jnp.float32),
        jnp.dot(fv0, wmv, preferred_element_type=jnp.float32),
        jnp.dot(fv1, wmv, preferred_element_type=jnp.float32),
        jnp.dot(fv2, wmv, preferred_element_type=jnp.float32),
    ], axis=1)


def _stage0(gn, d2, p):
    return pl.pallas_call(
        _stage0_body,
        out_shape=jax.ShapeDtypeStruct((M0, 4 * C), jnp.float32),
    )(gn, d2, p['Wr1_0'], p['Wr2_0'], p['W0_s'], p['W0_v'],
      p['Wms_0'], p['Wmv_0'])


def _stage1_body(fb_ref, idx_ref, d2_ref, wr1_ref, wr2_ref, wms_ref,
                 wmv_ref, wc1_ref, bc1_ref, wc2_ref, bc2_ref, wc3_ref,
                 bc3_ref, out_ref):
    fb = fb_ref[...]
    idx = idx_ref[...]
    oh = (idx == _iota2((1, M0), 1)).astype(jnp.float32)
    g = jnp.dot(oh, fb, preferred_element_type=jnp.float32)
    h = _rbf_h(d2_ref[...], R1, wr1_ref[...], wr2_ref[...])
    fs = _kmax(h * g[:, 0:C], M1)
    fv0 = _kmax(h * g[:, C:2 * C], M1)
    fv1 = _kmax(h * g[:, 2 * C:3 * C], M1)
    fv2 = _kmax(h * g[:, 3 * C:4 * C], M1)
    wms = wms_ref[...]
    wmv = wmv_ref[...]
    fs = jnp.dot(fs, wms, preferred_element_type=jnp.float32)
    fv0 = jnp.dot(fv0, wmv, preferred_element_type=jnp.float32)
    fv1 = jnp.dot(fv1, wmv, preferred_element_type=jnp.float32)
    fv2 = jnp.dot(fv2, wmv, preferred_element_type=jnp.float32)
    vn = jnp.sqrt((fv0 * fv0 + fv1 * fv1) + fv2 * fv2)
    inv = jnp.concatenate([fs, vn], axis=1)
    gmax = jnp.max(inv, axis=0, keepdims=True)
    h1 = _silu(jnp.dot(gmax, wc1_ref[...],
                       preferred_element_type=jnp.float32) + bc1_ref[...])
    m = jnp.mean(h1, axis=1, keepdims=True)
    v = jnp.mean((h1 - m) * (h1 - m), axis=1, keepdims=True)
    h1 = (h1 - m) / jnp.sqrt(v + 1e-5)
    h2 = _silu(jnp.dot(h1, wc2_ref[...],
                       preferred_element_type=jnp.float32) + bc2_ref[...])
    m = jnp.mean(h2, axis=1, keepdims=True)
    v = jnp.mean((h2 - m) * (h2 - m), axis=1, keepdims=True)
    h2 = (h2 - m) / jnp.sqrt(v + 1e-5)
    out_ref[...] = jnp.dot(h2, wc3_ref[...],
                           preferred_element_type=jnp.float32) + bc3_ref[...]


def _stage1(fb, idx, d2, p):
    return pl.pallas_call(
        _stage1_body,
        out_shape=jax.ShapeDtypeStruct((1, 40), jnp.float32),
    )(fb, idx, d2, p['Wr1_1'], p['Wr2_1'], p['Wms_1'], p['Wmv_1'],
      p['Wc1'], p['bc1'].reshape(1, -1), p['Wc2'], p['bc2'].reshape(1, -1),
      p['Wc3'], p['bc3'].reshape(1, -1))


def kernel(pos, params):
    posT = pos.T
    px8 = posT[0].reshape(8, N // 8)
    py8 = posT[1].reshape(8, N // 8)
    pz8 = posT[2].reshape(8, N // 8)
    px1 = posT[0].reshape(1, N)
    py1 = posT[1].reshape(1, N)
    pz1 = posT[2].reshape(1, N)

    aug = _prep(pos)

    c0x, c0y, c0z = _fps(px8, py8, pz8, M0)
    idx0, d20 = _ball_query(
        c0x.reshape(M0, 1), c0y.reshape(M0, 1), c0z.reshape(M0, 1),
        px1, py1, pz1, R0, 128)

    idx0f = idx0.T.reshape(N)
    gn = _sc_gather(idx0f, aug)
    fb = _stage0(gn, d20.T.reshape(N, 1), params)

    c1x, c1y, c1z = _fps(c0x, c0y, c0z, M1)
    idx1, d21 = _ball_query(
        c1x.reshape(M1, 1), c1y.reshape(M1, 1), c1z.reshape(M1, 1),
        c0x.reshape(1, M0), c0y.reshape(1, M0), c0z.reshape(1, M0),
        R1, 128)

    out = _stage1(fb, idx1.T.reshape(K * M1, 1), d21.T.reshape(K * M1, 1),
                  params)
    return out.reshape(40)


# speedup vs baseline: 1.1322x; 1.0433x over previous
import functools

import jax
import jax.numpy as jnp
from jax import lax
from jax.experimental import pallas as pl
from jax.experimental.pallas import tpu as pltpu
from jax.experimental.pallas import tpu_sc as plsc

N = 8192
C = 32
NUM_RBF = 32
M0 = 512
M1 = 128
K = 16
R0 = 0.2
R1 = 0.4

_BIGF = 1e10
_REMOVED = 3e10


def _iota2(shape, dim):
    return lax.broadcasted_iota(jnp.int32, shape, dim)


def _silu(x):
    return x * (1.0 / (1.0 + jnp.exp(-x)))


def _prep_body(pos_ref, aug_ref):
    p = pos_ref[...]
    x = p[:, 0:1]
    y = p[:, 1:2]
    z = p[:, 2:3]
    s0 = jnp.sqrt((x * x + y * y) + z * z)
    aug_ref[...] = jnp.concatenate(
        [s0, x, y, z, jnp.zeros((N, 124), jnp.float32)], axis=1)


def _prep(pos):
    return pl.pallas_call(
        _prep_body,
        out_shape=jax.ShapeDtypeStruct((N, 128), jnp.float32),
    )(pos)


def _fps_body(npts, nsel, px_ref, py_ref, pz_ref, ox_ref, oy_ref, oz_ref):
    px = px_ref[...]
    py = py_ref[...]
    pz = pz_ref[...]
    shape = px.shape
    cols = shape[1]
    lin = _iota2(shape, 0) * cols + _iota2(shape, 1)
    oshape = (8, nsel // 8)
    ocols = oshape[1]
    olin = _iota2(oshape, 0) * ocols + _iota2(oshape, 1)

    cat = jnp.concatenate([px, py, pz], axis=0)
    zc = jnp.zeros(cat.shape, jnp.float32)

    def step(i, carry):
        dist, cx, cy, cz, ax, ay, az = carry
        slot = olin == i
        ax = jnp.where(slot, cx, ax)
        ay = jnp.where(slot, cy, ay)
        az = jnp.where(slot, cz, az)
        dx = px - cx
        dy = py - cy
        dz = pz - cz
        d2 = (dx * dx + dy * dy) + dz * dz
        dist = jnp.minimum(dist, d2)
        mx = jnp.max(dist, axis=1, keepdims=True)
        mx = jnp.max(mx, axis=0, keepdims=True)
        t = jnp.where(dist == mx, lin, npts)
        t = jnp.min(t, axis=1, keepdims=True)
        cur = jnp.min(t, axis=0, keepdims=True)
        oh = lin == cur
        oh3 = jnp.concatenate([oh, oh, oh], axis=0)
        s1 = jnp.sum(jnp.where(oh3, cat, zc), axis=1, keepdims=True)
        cx = jnp.sum(s1[0:8], axis=0, keepdims=True)
        cy = jnp.sum(s1[8:16], axis=0, keepdims=True)
        cz = jnp.sum(s1[16:24], axis=0, keepdims=True)
        return dist, cx, cy, cz, ax, ay, az

    def body(i, carry):
        carry = step(4 * i, carry)
        carry = step(4 * i + 1, carry)
        carry = step(4 * i + 2, carry)
        return step(4 * i + 3, carry)

    dist0 = jnp.full(shape, 1e30, jnp.float32)
    c0x = px[0:1, 0:1]
    c0y = py[0:1, 0:1]
    c0z = pz[0:1, 0:1]
    zo = jnp.zeros(oshape, jnp.float32)
    _, _, _, _, ax, ay, az = lax.fori_loop(
        0, nsel // 4, body, (dist0, c0x, c0y, c0z, zo, zo, zo))
    ox_ref[...] = ax
    oy_ref[...] = ay
    oz_ref[...] = az


def _fps(px, py, pz, nsel):
    npts = px.shape[0] * px.shape[1]
    f = jax.ShapeDtypeStruct((8, nsel // 8), jnp.float32)
    return pl.pallas_call(
        functools.partial(_fps_body, npts, nsel),
        out_shape=(f, f, f),
    )(px, py, pz)


def _bq_body(npts, r2, cx_ref, cy_ref, cz_ref, px_ref, py_ref, pz_ref,
             idx_ref, d2_ref):
    cx = cx_ref[...]
    cy = cy_ref[...]
    cz = cz_ref[...]
    px = px_ref[...]
    py = py_ref[...]
    pz = pz_ref[...]
    dx = cx - px
    dy = cy - py
    dz = cz - pz
    d2 = (dx * dx + dy * dy) + dz * dz
    d = jnp.where(d2 <= r2, d2, _BIGF)
    nb = cx.shape[0]
    lin = _iota2((nb, npts), 1)
    lane = _iota2((nb, K), 1)
    vacc = jnp.zeros((nb, K), jnp.float32)
    iacc = jnp.zeros((nb, K), jnp.int32)
    for e in range(K):
        m = jnp.min(d, axis=1, keepdims=True)
        t = jnp.where(d == m, lin, npts)
        ix = jnp.min(t, axis=1, keepdims=True)
        vacc = jnp.where(lane == e, m, vacc)
        iacc = jnp.where(lane == e, ix, iacc)
        d = jnp.where(lin == ix, _REMOVED, d)
    valid = vacc < 1e9
    i0 = iacc[:, 0:1]
    v0 = vacc[:, 0:1]
    idx_ref[...] = jnp.where(valid, iacc, i0)
    d2_ref[...] = jnp.where(valid, vacc, v0)


def _ball_query(cx, cy, cz, px, py, pz, radius, block):
    nc = cx.shape[0]
    npts = px.shape[1]
    grid = nc // block
    cspec = pl.BlockSpec((block, 1), lambda b: (b, 0))
    pspec = pl.BlockSpec((1, npts), lambda b: (0, 0))
    ospec = pl.BlockSpec((block, K), lambda b: (b, 0))
    return pl.pallas_call(
        functools.partial(_bq_body, npts, radius * radius),
        grid=(grid,),
        in_specs=[cspec, cspec, cspec, pspec, pspec, pspec],
        out_specs=(ospec, ospec),
        out_shape=(jax.ShapeDtypeStruct((nc, K), jnp.int32),
                   jax.ShapeDtypeStruct((nc, K), jnp.float32)),
    )(cx, cy, cz, px, py, pz)


_SC_NC = 2
_SC_NS = 16
_SC_NW = _SC_NC * _SC_NS
_SC_BPW = N // _SC_NW


def _sc_gather_body(idx_hbm, tab_hbm, out_hbm, idx_v, rows_v, sem):
    wid = lax.axis_index("s") * _SC_NC + lax.axis_index("c")
    base = wid * _SC_BPW
    pltpu.sync_copy(idx_hbm.at[pl.ds(base, _SC_BPW)], idx_v)
    pltpu.async_copy(tab_hbm.at[idx_v], rows_v, sem).wait()
    pltpu.sync_copy(rows_v, out_hbm.at[pl.ds(base, _SC_BPW)])


def _sc_gather(idx, tab):
    mesh = plsc.VectorSubcoreMesh(core_axis_name="c", subcore_axis_name="s")
    k = pl.kernel(
        _sc_gather_body,
        mesh=mesh,
        out_type=jax.ShapeDtypeStruct((N, 128), jnp.float32),
        scratch_types=[
            pltpu.VMEM((_SC_BPW,), jnp.int32),
            pltpu.VMEM((_SC_BPW, 128), jnp.float32),
            pltpu.SemaphoreType.DMA,
        ],
    )
    return k(idx, tab)


def _rbf_h(d2, cutoff, wr1, wr2):
    dist = jnp.maximum(jnp.sqrt(d2), 1e-8)
    mu = _iota2((1, NUM_RBF), 1).astype(jnp.float32) * (
        cutoff / (NUM_RBF - 1))
    beta = (NUM_RBF / cutoff) ** 2
    dd = dist - mu
    rbf = jnp.exp(-beta * (dd * dd))
    a = _silu(jnp.dot(rbf, wr1, preferred_element_type=jnp.float32))
    return jnp.dot(a, wr2, preferred_element_type=jnp.float32)


def _kmax(t, m):
    acc = t[0:m, :]
    for k in range(1, K):
        acc = jnp.maximum(acc, t[k * m:(k + 1) * m, :])
    return acc


def _stage0_body(g_ref, d2_ref, wr1_ref, wr2_ref, w0s_ref, w0v_ref,
                 wms_ref, wmv_ref, out_ref):
    g = g_ref[...]
    h = _rbf_h(d2_ref[...], R0, wr1_ref[...], wr2_ref[...])
    w0s = w0s_ref[...]
    w0v = w0v_ref[...]
    fs = _kmax(h * g[:, 0:1] * w0s, M0)
    fv0 = _kmax(h * g[:, 1:2] * w0v, M0)
    fv1 = _kmax(h * g[:, 2:3] * w0v, M0)
    fv2 = _kmax(h * g[:, 3:4] * w0v, M0)
    wms = wms_ref[...]
    wmv = wmv_ref[...]
    out_ref[...] = jnp.concatenate([
        jnp.dot(fs, wms, preferred_element_type=jnp.float32),
        jnp.dot(fv0, wmv, preferred_element_type=jnp.float32),
        jnp.dot(fv1, wmv, preferred_element_type=jnp.float32),
        jnp.dot(fv2, wmv, preferred_element_type=jnp.float32),
    ], axis=1)


def _stage0(gn, d2, p):
    return pl.pallas_call(
        _stage0_body,
        out_shape=jax.ShapeDtypeStruct((M0, 4 * C), jnp.float32),
    )(gn, d2, p['Wr1_0'], p['Wr2_0'], p['W0_s'], p['W0_v'],
      p['Wms_0'], p['Wmv_0'])


def _stage1_body(fb_ref, idx_ref, d2_ref, wr1_ref, wr2_ref, wms_ref,
                 wmv_ref, wc1_ref, bc1_ref, wc2_ref, bc2_ref, wc3_ref,
                 bc3_ref, out_ref):
    fb = fb_ref[...]
    idx = idx_ref[...]
    oh = (idx == _iota2((1, M0), 1)).astype(jnp.float32)
    g = jnp.dot(oh, fb, preferred_element_type=jnp.float32)
    h = _rbf_h(d2_ref[...], R1, wr1_ref[...], wr2_ref[...])
    fs = _kmax(h * g[:, 0:C], M1)
    fv0 = _kmax(h * g[:, C:2 * C], M1)
    fv1 = _kmax(h * g[:, 2 * C:3 * C], M1)
    fv2 = _kmax(h * g[:, 3 * C:4 * C], M1)
    wms = wms_ref[...]
    wmv = wmv_ref[...]
    fs = jnp.dot(fs, wms, preferred_element_type=jnp.float32)
    fv0 = jnp.dot(fv0, wmv, preferred_element_type=jnp.float32)
    fv1 = jnp.dot(fv1, wmv, preferred_element_type=jnp.float32)
    fv2 = jnp.dot(fv2, wmv, preferred_element_type=jnp.float32)
    vn = jnp.sqrt((fv0 * fv0 + fv1 * fv1) + fv2 * fv2)
    inv = jnp.concatenate([fs, vn], axis=1)
    gmax = jnp.max(inv, axis=0, keepdims=True)
    h1 = _silu(jnp.dot(gmax, wc1_ref[...],
                       preferred_element_type=jnp.float32) + bc1_ref[...])
    m = jnp.mean(h1, axis=1, keepdims=True)
    v = jnp.mean((h1 - m) * (h1 - m), axis=1, keepdims=True)
    h1 = (h1 - m) / jnp.sqrt(v + 1e-5)
    h2 = _silu(jnp.dot(h1, wc2_ref[...],
                       preferred_element_type=jnp.float32) + bc2_ref[...])
    m = jnp.mean(h2, axis=1, keepdims=True)
    v = jnp.mean((h2 - m) * (h2 - m), axis=1, keepdims=True)
    h2 = (h2 - m) / jnp.sqrt(v + 1e-5)
    out_ref[...] = jnp.dot(h2, wc3_ref[...],
                           preferred_element_type=jnp.float32) + bc3_ref[...]


def _stage1(fb, idx, d2, p):
    return pl.pallas_call(
        _stage1_body,
        out_shape=jax.ShapeDtypeStruct((1, 40), jnp.float32),
    )(fb, idx, d2, p['Wr1_1'], p['Wr2_1'], p['Wms_1'], p['Wmv_1'],
      p['Wc1'], p['bc1'].reshape(1, -1), p['Wc2'], p['bc2'].reshape(1, -1),
      p['Wc3'], p['bc3'].reshape(1, -1))


def kernel(pos, params):
    posT = pos.T
    px8 = posT[0].reshape(8, N // 8)
    py8 = posT[1].reshape(8, N // 8)
    pz8 = posT[2].reshape(8, N // 8)
    px1 = posT[0].reshape(1, N)
    py1 = posT[1].reshape(1, N)
    pz1 = posT[2].reshape(1, N)

    aug = _prep(pos)

    c0x, c0y, c0z = _fps(px8, py8, pz8, M0)
    idx0, d20 = _ball_query(
        c0x.reshape(M0, 1), c0y.reshape(M0, 1), c0z.reshape(M0, 1),
        px1, py1, pz1, R0, 128)

    idx0f = idx0.T.reshape(N)
    gn = _sc_gather(idx0f, aug)
    fb = _stage0(gn, d20.T.reshape(N, 1), params)

    c1x, c1y, c1z = _fps(c0x, c0y, c0z, M1)
    idx1, d21 = _ball_query(
        c1x.reshape(M1, 1), c1y.reshape(M1, 1), c1z.reshape(M1, 1),
        c0x.reshape(1, M0), c0y.reshape(1, M0), c0z.reshape(1, M0),
        R1, 128)

    out = _stage1(fb, idx1.T.reshape(K * M1, 1), d21.T.reshape(K * M1, 1),
                  params)
    return out.reshape(40)
